# baseline (speedup 1.0000x reference)
import functools

import jax
import jax.numpy as jnp
from jax.experimental import pallas as pl
from jax.experimental.pallas import tpu as pltpu

_LOSS_COEF = 0.01


def _cv_sq(v, n):
    mean = jnp.sum(v, axis=1, keepdims=True) / n
    var = jnp.sum((v - mean) ** 2, axis=1, keepdims=True) / (n - 1)
    return var / (mean * mean + 1e-10)


def _gating(xs, sw, sb0, wg, sbk, ne):
    nl, nd = xs.shape[1], xs.shape[2]
    g_in = jnp.dot(xs.reshape(sbk * nl, nd), sw,
                   preferred_element_type=jnp.float32)
    g_in = (g_in + sb0).reshape(sbk, nl)
    logits = jnp.dot(g_in, wg,
                     preferred_element_type=jnp.float32)

    iota = jax.lax.broadcasted_iota(jnp.int32, (sbk, ne), 1)
    m1 = jnp.max(logits, axis=1, keepdims=True)
    idx1 = jnp.min(jnp.where(logits == m1, iota, ne), axis=1, keepdims=True)
    sel1 = iota == idx1
    masked = jnp.where(sel1, -jnp.inf, logits)
    m2 = jnp.max(masked, axis=1, keepdims=True)
    idx2 = jnp.min(jnp.where(masked == m2, iota, ne), axis=1, keepdims=True)
    sel2 = iota == idx2
    t = jnp.exp(m2 - m1)
    g1 = 1.0 / (1.0 + t)
    g2 = t / (1.0 + t)
    gates = jnp.where(sel1, g1, 0.0) + jnp.where(sel2, g2, 0.0)
    return idx1, idx2, g1, g2, gates


def _body(x_ref, sw_ref, sb_ref, wg_ref, ewt_ref,
          out_ref, loss_ref, imp_ref, load_ref, *, bb, sbk, ne, nsteps):
    i = pl.program_id(0)
    sw = sw_ref[...]
    sb0 = sb_ref[...]
    wg = wg_ref[...]

    @pl.when(i == 0)
    def _():
        imp_ref[...] = jnp.zeros_like(imp_ref)
        load_ref[...] = jnp.zeros_like(load_ref)

    for h in range(bb // sbk):
        xs = x_ref[h * sbk:(h + 1) * sbk]
        nd = xs.shape[2]
        idx1, idx2, g1, g2, gates = _gating(xs, sw, sb0, wg, sbk, ne)

        imp_ref[...] = imp_ref[...] + jnp.sum(gates, axis=0, keepdims=True)
        load_ref[...] = load_ref[...] + jnp.sum(
            (gates > 0.0).astype(jnp.float32), axis=0, keepdims=True)

        xaug = jnp.concatenate(
            [xs, jnp.ones((sbk, 1, nd), jnp.float32)], axis=1)

        for b in range(sbk):
            ew1 = ewt_ref[idx1[b, 0]]
            ew2 = ewt_ref[idx2[b, 0]]
            gwt_b = ew1 * g1[b:b + 1, 0:1] + ew2 * g2[b:b + 1, 0:1]
            out_ref[h * sbk + b] = jnp.dot(
                gwt_b, xaug[b],
                preferred_element_type=jnp.float32)

    @pl.when(i == nsteps - 1)
    def _():
        loss_ref[...] = _LOSS_COEF * (
            _cv_sq(imp_ref[...], ne) + _cv_sq(load_ref[...], ne))


def kernel(x, x_mark_enc, start_w, start_b, w_gate, expert_w, expert_b):
    del x_mark_enc
    B, L, D = x.shape
    E = w_gate.shape[1]
    P = expert_w.shape[2]
    BB = 64
    nsteps = B // BB

    sw = start_w.reshape(D, 1)
    sb = start_b.reshape(1, 1)
    ewt = jnp.concatenate(
        [jnp.swapaxes(expert_w, 1, 2), expert_b[:, :, None]],
        axis=2)

    out, loss = pl.pallas_call(
        functools.partial(_body, bb=BB, sbk=16, ne=E, nsteps=nsteps),
        grid=(nsteps,),
        in_specs=[
            pl.BlockSpec((BB, L, D), lambda i: (i, 0, 0)),
            pl.BlockSpec((D, 1), lambda i: (0, 0)),
            pl.BlockSpec((1, 1), lambda i: (0, 0)),
            pl.BlockSpec((L, E), lambda i: (0, 0)),
            pl.BlockSpec((E, P, L + 1), lambda i: (0, 0, 0)),
        ],
        out_specs=[
            pl.BlockSpec((BB, P, D), lambda i: (i, 0, 0)),
            pl.BlockSpec((1, 1), lambda i: (0, 0)),
        ],
        out_shape=[
            jax.ShapeDtypeStruct((B, P, D), jnp.float32),
            jax.ShapeDtypeStruct((1, 1), jnp.float32),
        ],
        scratch_shapes=[
            pltpu.VMEM((1, E), jnp.float32),
            pltpu.VMEM((1, E), jnp.float32),
        ],
    )(x, sw, sb, w_gate, ewt)

    return out, loss[0, 0], jnp.float32(0.0)


# speedup vs baseline: 1.1573x; 1.1573x over previous
import functools

import jax
import jax.numpy as jnp
from jax.experimental import pallas as pl
from jax.experimental.pallas import tpu as pltpu

_LOSS_COEF = 0.01


def _cv_sq(v, n):
    mean = jnp.sum(v, axis=1, keepdims=True) / n
    var = jnp.sum((v - mean) ** 2, axis=1, keepdims=True) / (n - 1)
    return var / (mean * mean + 1e-10)


def _gating(xs, sw, sb0, wg, sbk, ne):
    nl, nd = xs.shape[1], xs.shape[2]
    g_in = jnp.dot(xs.reshape(sbk * nl, nd), sw,
                   preferred_element_type=jnp.float32)
    g_in = (g_in + sb0).reshape(sbk, nl)
    logits = jnp.dot(g_in, wg,
                     preferred_element_type=jnp.float32)

    iota = jax.lax.broadcasted_iota(jnp.int32, (sbk, ne), 1)
    m1 = jnp.max(logits, axis=1, keepdims=True)
    idx1 = jnp.min(jnp.where(logits == m1, iota, ne), axis=1, keepdims=True)
    sel1 = iota == idx1
    masked = jnp.where(sel1, -jnp.inf, logits)
    m2 = jnp.max(masked, axis=1, keepdims=True)
    idx2 = jnp.min(jnp.where(masked == m2, iota, ne), axis=1, keepdims=True)
    sel2 = iota == idx2
    t = jnp.exp(m2 - m1)
    g1 = 1.0 / (1.0 + t)
    g2 = t / (1.0 + t)
    gates = jnp.where(sel1, g1, 0.0) + jnp.where(sel2, g2, 0.0)
    return idx1, idx2, g1, g2, gates


def _body(x_ref, sw_ref, sb_ref, wg_ref, ewt_ref,
          out_ref, loss_ref, imp_ref, load_ref, *, bb, sbk, ne, nsteps):
    i = pl.program_id(0)
    sw = sw_ref[...]
    sb0 = sb_ref[...]
    wg = wg_ref[...]

    @pl.when(i == 0)
    def _():
        imp_ref[...] = jnp.zeros_like(imp_ref)
        load_ref[...] = jnp.zeros_like(load_ref)

    sub = []
    for h in range(bb // sbk):
        xs = x_ref[h * sbk:(h + 1) * sbk]
        nd = xs.shape[2]
        idx1, idx2, g1, g2, gates = _gating(xs, sw, sb0, wg, sbk, ne)

        imp_ref[...] = imp_ref[...] + jnp.sum(gates, axis=0, keepdims=True)
        load_ref[...] = load_ref[...] + jnp.sum(
            (gates > 0.0).astype(jnp.float32), axis=0, keepdims=True)

        xaug = jnp.concatenate(
            [xs, jnp.ones((sbk, 1, nd), jnp.float32)], axis=1)
        sub.append((idx1, idx2, g1, g2, xaug))

    for h, (idx1, idx2, g1, g2, xaug) in enumerate(sub):
        for b in range(sbk):
            ew1 = ewt_ref[idx1[b, 0]]
            ew2 = ewt_ref[idx2[b, 0]]
            gwt_b = ew1 * g1[b:b + 1, 0:1] + ew2 * g2[b:b + 1, 0:1]
            out_ref[h * sbk + b] = jnp.dot(
                gwt_b, xaug[b],
                preferred_element_type=jnp.float32)

    @pl.when(i == nsteps - 1)
    def _():
        loss_ref[...] = _LOSS_COEF * (
            _cv_sq(imp_ref[...], ne) + _cv_sq(load_ref[...], ne))


def kernel(x, x_mark_enc, start_w, start_b, w_gate, expert_w, expert_b):
    del x_mark_enc
    B, L, D = x.shape
    E = w_gate.shape[1]
    P = expert_w.shape[2]
    BB = 64
    nsteps = B // BB

    sw = start_w.reshape(D, 1)
    sb = start_b.reshape(1, 1)
    ewt = jnp.concatenate(
        [jnp.swapaxes(expert_w, 1, 2), expert_b[:, :, None]],
        axis=2)

    out, loss = pl.pallas_call(
        functools.partial(_body, bb=BB, sbk=32, ne=E, nsteps=nsteps),
        grid=(nsteps,),
        in_specs=[
            pl.BlockSpec((BB, L, D), lambda i: (i, 0, 0)),
            pl.BlockSpec((D, 1), lambda i: (0, 0)),
            pl.BlockSpec((1, 1), lambda i: (0, 0)),
            pl.BlockSpec((L, E), lambda i: (0, 0)),
            pl.BlockSpec((E, P, L + 1), lambda i: (0, 0, 0)),
        ],
        out_specs=[
            pl.BlockSpec((BB, P, D), lambda i: (i, 0, 0)),
            pl.BlockSpec((1, 1), lambda i: (0, 0)),
        ],
        out_shape=[
            jax.ShapeDtypeStruct((B, P, D), jnp.float32),
            jax.ShapeDtypeStruct((1, 1), jnp.float32),
        ],
        scratch_shapes=[
            pltpu.VMEM((1, E), jnp.float32),
            pltpu.VMEM((1, E), jnp.float32),
        ],
    )(x, sw, sb, w_gate, ewt)

    return out, loss[0, 0], jnp.float32(0.0)
